# baseline (speedup 1.0000x reference)
import jax
import jax.numpy as jnp
from jax import lax
from jax.experimental import pallas as pl
from jax.experimental.pallas import tpu as pltpu
from jax.experimental.pallas import tpu_sc as plsc

IN_F = 6
OUT_F = 11
LANES = 16
IMG_SIZE = 128


def _make_body(nc, rows_per_worker, n_rows):
    def body(x_hbm, out_hbm, in_v, out_v, sem):
        wid = lax.axis_index("s") * nc + lax.axis_index("c")
        row0 = wid * rows_per_worker

        in_copies = [
            pltpu.async_copy(
                x_hbm.at[pl.ds(f * n_rows + row0, rows_per_worker)],
                in_v.at[pl.ds(f * rows_per_worker, rows_per_worker)],
                sem,
            )
            for f in range(IN_F)
        ]
        for c in in_copies:
            c.wait()

        inv = jnp.float32(1.0 / IMG_SIZE)
        zero = jnp.zeros((LANES,), jnp.float32)

        @plsc.parallel_loop(0, rows_per_worker, LANES, unroll=8)
        def _chunk(o):
            g = [in_v[pl.ds(f * rows_per_worker + o, LANES)] for f in range(IN_F)]
            prob = g[4]
            cid = g[5].astype(jnp.int32)
            cid = jnp.minimum(jnp.maximum(cid, 0), 8)
            c = (cid >= 3).astype(jnp.int32) + (cid >= 6).astype(jnp.int32)
            s = cid - 3 * c
            outs = (
                g[0] * inv, g[1] * inv, g[2] * inv, g[3] * inv,
                jnp.where(c == 0, prob, zero),
                jnp.where(c == 1, prob, zero),
                jnp.where(c == 2, prob, zero),
                jnp.where(s == 0, prob, zero),
                jnp.where(s == 1, prob, zero),
                jnp.where(s == 2, prob, zero),
                prob,
            )
            for f in range(OUT_F):
                out_v[pl.ds(f * rows_per_worker + o, LANES)] = outs[f]

        out_copies = [
            pltpu.async_copy(
                out_v.at[pl.ds(f * rows_per_worker, rows_per_worker)],
                out_hbm.at[pl.ds(f * n_rows + row0, rows_per_worker)],
                sem,
            )
            for f in range(OUT_F)
        ]
        for c in out_copies:
            c.wait()

    return body


def kernel(x):
    n, obj_num, feat = x.shape
    rows = n * obj_num
    mesh = plsc.VectorSubcoreMesh(core_axis_name="c", subcore_axis_name="s")
    nw = mesh.num_cores * mesh.num_subcores
    rows_per_worker = rows // nw

    k = pl.kernel(
        _make_body(mesh.num_cores, rows_per_worker, rows),
        out_type=jax.ShapeDtypeStruct((rows * OUT_F,), jnp.float32),
        mesh=mesh,
        compiler_params=pltpu.CompilerParams(needs_layout_passes=False),
        scratch_types=[
            pltpu.VMEM((rows_per_worker * IN_F,), jnp.float32),
            pltpu.VMEM((rows_per_worker * OUT_F,), jnp.float32),
            pltpu.SemaphoreType.DMA,
        ],
    )
    xt = jnp.transpose(x, (2, 1, 0)).reshape(-1)
    out_flat = k(xt)
    return jnp.transpose(out_flat.reshape(OUT_F, obj_num, n), (2, 1, 0))


# speedup vs baseline: 1.0528x; 1.0528x over previous
import jax
import jax.numpy as jnp
from jax import lax
from jax.experimental import pallas as pl
from jax.experimental.pallas import tpu as pltpu
from jax.experimental.pallas import tpu_sc as plsc

IN_F = 6
OUT_F = 11
LANES = 16
IMG_SIZE = 128


def _make_body(nc, rows_per_worker, n_rows):
    def body(x_hbm, out_hbm, in_v, out_v, sem):
        wid = lax.axis_index("s") * nc + lax.axis_index("c")
        row0 = wid * rows_per_worker

        in_copies = [
            pltpu.async_copy(
                x_hbm.at[pl.ds(f * n_rows + row0, rows_per_worker)],
                in_v.at[pl.ds(f * rows_per_worker, rows_per_worker)],
                sem,
            )
            for f in range(IN_F)
        ]
        for c in in_copies:
            c.wait()

        inv = jnp.float32(1.0 / IMG_SIZE)
        zero = jnp.zeros((LANES,), jnp.float32)

        @plsc.parallel_loop(0, rows_per_worker, LANES, unroll=2)
        def _chunk(o):
            g = [in_v[pl.ds(f * rows_per_worker + o, LANES)] for f in range(IN_F)]
            prob = g[4]
            cid = g[5].astype(jnp.int32)
            cid = jnp.minimum(jnp.maximum(cid, 0), 8)
            c = (cid >= 3).astype(jnp.int32) + (cid >= 6).astype(jnp.int32)
            s = cid - 3 * c
            outs = (
                g[0] * inv, g[1] * inv, g[2] * inv, g[3] * inv,
                jnp.where(c == 0, prob, zero),
                jnp.where(c == 1, prob, zero),
                jnp.where(c == 2, prob, zero),
                jnp.where(s == 0, prob, zero),
                jnp.where(s == 1, prob, zero),
                jnp.where(s == 2, prob, zero),
                prob,
            )
            for f in range(OUT_F):
                out_v[pl.ds(f * rows_per_worker + o, LANES)] = outs[f]

        out_copies = [
            pltpu.async_copy(
                out_v.at[pl.ds(f * rows_per_worker, rows_per_worker)],
                out_hbm.at[pl.ds(f * n_rows + row0, rows_per_worker)],
                sem,
            )
            for f in range(OUT_F)
        ]
        for c in out_copies:
            c.wait()

    return body


def kernel(x):
    n, obj_num, feat = x.shape
    rows = n * obj_num
    mesh = plsc.VectorSubcoreMesh(core_axis_name="c", subcore_axis_name="s")
    nw = mesh.num_cores * mesh.num_subcores
    rows_per_worker = rows // nw

    k = pl.kernel(
        _make_body(mesh.num_cores, rows_per_worker, rows),
        out_type=jax.ShapeDtypeStruct((rows * OUT_F,), jnp.float32),
        mesh=mesh,
        compiler_params=pltpu.CompilerParams(needs_layout_passes=False),
        scratch_types=[
            pltpu.VMEM((rows_per_worker * IN_F,), jnp.float32),
            pltpu.VMEM((rows_per_worker * OUT_F,), jnp.float32),
            pltpu.SemaphoreType.DMA,
        ],
    )
    xt = jnp.transpose(x, (2, 1, 0)).reshape(-1)
    out_flat = k(xt)
    return jnp.transpose(out_flat.reshape(OUT_F, obj_num, n), (2, 1, 0))


# speedup vs baseline: 1.0568x; 1.0038x over previous
import jax
import jax.numpy as jnp
from jax import lax
from jax.experimental import pallas as pl
from jax.experimental.pallas import tpu as pltpu
from jax.experimental.pallas import tpu_sc as plsc

IN_F = 6
OUT_F = 11
LANES = 16
IMG_SIZE = 128


def _make_body(nc, rows_per_worker, n_rows):
    def body(x_hbm, out_hbm, in_v, out_v, sem):
        wid = lax.axis_index("s") * nc + lax.axis_index("c")
        row0 = wid * rows_per_worker

        in_copies = [
            pltpu.async_copy(
                x_hbm.at[pl.ds(f * n_rows + row0, rows_per_worker)],
                in_v.at[pl.ds(f * rows_per_worker, rows_per_worker)],
                sem,
            )
            for f in range(IN_F)
        ]
        for c in in_copies:
            c.wait()

        inv = jnp.float32(1.0 / IMG_SIZE)
        zero = jnp.zeros((LANES,), jnp.float32)

        @plsc.parallel_loop(0, rows_per_worker, LANES, unroll=1)
        def _chunk(o):
            g = [in_v[pl.ds(f * rows_per_worker + o, LANES)] for f in range(IN_F)]
            prob = g[4]
            cid = g[5].astype(jnp.int32)
            cid = jnp.minimum(jnp.maximum(cid, 0), 8)
            c = (cid >= 3).astype(jnp.int32) + (cid >= 6).astype(jnp.int32)
            s = cid - 3 * c
            outs = (
                g[0] * inv, g[1] * inv, g[2] * inv, g[3] * inv,
                jnp.where(c == 0, prob, zero),
                jnp.where(c == 1, prob, zero),
                jnp.where(c == 2, prob, zero),
                jnp.where(s == 0, prob, zero),
                jnp.where(s == 1, prob, zero),
                jnp.where(s == 2, prob, zero),
                prob,
            )
            for f in range(OUT_F):
                out_v[pl.ds(f * rows_per_worker + o, LANES)] = outs[f]

        out_copies = [
            pltpu.async_copy(
                out_v.at[pl.ds(f * rows_per_worker, rows_per_worker)],
                out_hbm.at[pl.ds(f * n_rows + row0, rows_per_worker)],
                sem,
            )
            for f in range(OUT_F)
        ]
        for c in out_copies:
            c.wait()

    return body


def kernel(x):
    n, obj_num, feat = x.shape
    rows = n * obj_num
    mesh = plsc.VectorSubcoreMesh(core_axis_name="c", subcore_axis_name="s")
    nw = mesh.num_cores * mesh.num_subcores
    rows_per_worker = rows // nw

    k = pl.kernel(
        _make_body(mesh.num_cores, rows_per_worker, rows),
        out_type=jax.ShapeDtypeStruct((rows * OUT_F,), jnp.float32),
        mesh=mesh,
        compiler_params=pltpu.CompilerParams(needs_layout_passes=False),
        scratch_types=[
            pltpu.VMEM((rows_per_worker * IN_F,), jnp.float32),
            pltpu.VMEM((rows_per_worker * OUT_F,), jnp.float32),
            pltpu.SemaphoreType.DMA,
        ],
    )
    xt = jnp.transpose(x, (2, 1, 0)).reshape(-1)
    out_flat = k(xt)
    return jnp.transpose(out_flat.reshape(OUT_F, obj_num, n), (2, 1, 0))
